# baseline (speedup 1.0000x reference)
import functools

import jax
import jax.numpy as jnp
from jax import lax
from jax.experimental import pallas as pl
from jax.experimental.pallas import tpu as pltpu
from jax.experimental.pallas import tpu_sc as plsc

EMBED_DIM = 64
BATCH = 16384

_info = plsc.get_sparse_core_info()
_NC, _NS = _info.num_cores, _info.num_subcores

_BLK = 512
_R_MAIN = 62464
_FLUSH = 64
_HPAD = BATCH + 64


def _make_sweep(num_rows, dim, batch):
    chalf = dim // _NC
    b_per_w = batch // _NS
    mesh = plsc.VectorSubcoreMesh(core_axis_name="c", subcore_axis_name="s")

    @functools.partial(
        pl.kernel,
        mesh=mesh,
        out_type=jax.ShapeDtypeStruct((_NC * _HPAD, 128), jnp.float32),
        scratch_types=[
            pltpu.VMEM((chalf, _BLK), jnp.float32),
            pltpu.VMEM((chalf, _BLK), jnp.float32),
            pltpu.VMEM((2048,), jnp.int32),
            pltpu.VMEM((batch + 16,), jnp.int32),
            pltpu.VMEM((batch + 16,), jnp.int32),
            pltpu.VMEM((batch + 16,), jnp.int32),
            pltpu.VMEM((batch + 16,), jnp.int32),
            pltpu.VMEM((_FLUSH, 128), jnp.float32),
            pltpu.VMEM((_FLUSH,), jnp.int32),
            pltpu.SemaphoreType.DMA,
            pltpu.SemaphoreType.DMA,
            pltpu.SemaphoreType.DMA,
        ],
        compiler_params=pltpu.CompilerParams(needs_layout_passes=False),
    )
    def sweep_kernel(idx_hbm, tt_hbm, tab_hbm, out_hbm, slab_a,
                     slab_b, idx_c, loc_r, loc_b, blk_r, blk_b, rows_p,
                     bpad, sem_a, sem_b, sem_s):
        core = lax.axis_index("c")
        sub = lax.axis_index("s")
        c0 = core * chalf
        tile_r0 = pl.multiple_of(sub * _R_MAIN, 128)
        is_last = sub == _NS - 1
        r_range = jnp.where(is_last, num_rows - (_NS - 1) * _R_MAIN,
                            _R_MAIN)
        hbase = core * _HPAD
        dump = hbase + batch + sub
        lane = lax.iota(jnp.int32, 16)
        pad_slot = jnp.int32(batch + 8)

        def compact2(ref1, x1, ref2, x2, m, cnt):
            cs = plsc.cumsum(m.astype(jnp.int32))
            pos = jnp.where(m, cnt + cs - 1, pad_slot)
            plsc.store_scatter(ref1, [pos], x1)
            plsc.store_scatter(ref2, [pos], x2)
            return cnt + cs[15]

        def fire(dma_r0, slab, sem):
            pltpu.async_copy(
                tt_hbm.at[pl.ds(c0, chalf),
                          pl.ds(pl.multiple_of(tile_r0 + dma_r0, 128),
                                _BLK)],
                slab, sem)

        def drain(slab, sem):
            pltpu.make_async_copy(
                tt_hbm.at[pl.ds(c0, chalf), pl.ds(tile_r0, _BLK)],
                slab, sem).wait()

        fire(0, slab_a, sem_a)

        def scan_chunk(ch, cnt):
            pltpu.sync_copy(idx_hbm.at[pl.ds(ch * 2048, 2048)], idx_c)

            def scan_vec(i, cnt):
                v = idx_c[pl.ds(i * 16, 16)]
                vr = v - tile_r0
                m = (vr >= 0) & (vr < r_range)
                b = ch * 2048 + i * 16 + lane
                return compact2(loc_r, vr, loc_b, b, m, cnt)

            return lax.fori_loop(0, 2048 // 16, scan_vec, cnt)

        cnt = lax.fori_loop(0, batch // 2048, scan_chunk, jnp.int32(0))

        def do_block(k, slab, sem, fire_next, process_fn):
            w0 = pl.multiple_of(k * _BLK, 128)
            drain(slab, sem)
            if fire_next is not None:
                fire_next()
            process_fn(w0, w0 + _BLK, w0, slab)

        def process(w0, w1, dma_r0, slab):

            def rescan(i, bc):
                vr = loc_r[pl.ds(i * 16, 16)]
                m = (vr >= w0) & (vr < w1)
                vb = loc_b[pl.ds(i * 16, 16)]
                return compact2(blk_r, vr - dma_r0, blk_b, vb, m, bc)

            nvec = (cnt + 15) // 16
            bc = lax.fori_loop(0, nvec, rescan, jnp.int32(0))

            def flush(sb, carry):
                base = sb * _FLUSH
                nit = jnp.minimum(_FLUSH, bc - base)

                def fill(i, carry):
                    it = base + i
                    rr = blk_r[pl.ds(it, 16)][0]
                    for g in range(chalf // 16):
                        rows_p[i, pl.ds(g * 16, 16)] = plsc.load_gather(
                            slab, [lane + g * 16, jnp.full((16,), rr,
                                                           jnp.int32)])
                    return carry

                lax.fori_loop(0, nit, fill, 0)
                for g in range(_FLUSH // 16):
                    li = base + g * 16 + lane
                    vb = blk_b[pl.ds(base + g * 16, 16)]
                    bpad[pl.ds(g * 16, 16)] = jnp.where(
                        li < bc, vb + hbase, dump)
                pltpu.async_copy(rows_p, out_hbm.at[bpad], sem_s).wait()
                return carry

            lax.fori_loop(0, (bc + _FLUSH - 1) // _FLUSH, flush, 0)

        def pair_loop(p, carry):
            k = p * 2
            do_block(k, slab_a, sem_a,
                     lambda: fire(pl.multiple_of((k + 1) * _BLK, 128),
                                  slab_b, sem_b),
                     process)
            do_block(k + 1, slab_b, sem_b,
                     lambda: fire(pl.multiple_of((k + 2) * _BLK, 128),
                                  slab_a, sem_a),
                     process)
            return carry

        lax.fori_loop(0, _R_MAIN // _BLK // 2, pair_loop, 0)
        drain(slab_a, sem_a)

        @pl.when(is_last)
        def _tail():
            process(122 * _BLK, 123 * _BLK, 122 * _BLK, slab_a)
            def rescan_t(i, bc):
                vr = loc_r[pl.ds(i * 16, 16)]
                m = (vr >= 123 * _BLK) & (vr < r_range)
                vb = loc_b[pl.ds(i * 16, 16)]
                return compact2(blk_r, vr, blk_b, vb, m, bc)

            bc = lax.fori_loop(0, (cnt + 15) // 16, rescan_t, jnp.int32(0))

            def tail_item(i, carry):
                rr = blk_r[pl.ds(i, 16)][0]
                bb = blk_b[pl.ds(i, 16)][0]
                pltpu.sync_copy(
                    tab_hbm.at[pl.ds(tile_r0 + rr, 1), pl.ds(c0, chalf)],
                    rows_p.at[pl.ds(0, 1), pl.ds(0, chalf)])
                pltpu.sync_copy(
                    rows_p.at[pl.ds(0, 1)],
                    out_hbm.at[pl.ds(hbase + bb, 1)])
                return carry

            lax.fori_loop(0, bc, tail_item, 0)

    return sweep_kernel


@jax.jit
def kernel(user_inputs, table):
    num_rows, dim = table.shape
    batch = user_inputs.shape[0]
    sweep = _make_sweep(num_rows, dim, batch)
    y2 = sweep(user_inputs.astype(jnp.int32), table.T, table)
    return jnp.concatenate(
        [y2[:batch, : dim // 2], y2[_HPAD:_HPAD + batch, : dim // 2]],
        axis=1)
